# baseline (speedup 1.0000x reference)
import functools

import jax
import jax.numpy as jnp
from jax.experimental import pallas as pl
from jax.experimental.pallas import tpu as pltpu

_MIB = 1024 * 1024
_LANES = 128


def _fused_body(xoa_ref, xob_ref, xe_ref, xx_ref,
                w1_ref, b1_ref, w2_ref, b2_ref, w3_ref, b3_ref,
                out_ref,
                osa_sum, osa_max, osb_sum, osb_max,
                es_sum, es_max, xs_sum, xs_max,
                *, nco, ns, bloc, widths, inv_o, inv_e, inv_x):
    k = pl.program_id(1)

    def _big(ref, s_sc, m_sc):
        x = ref[...]
        s = jnp.sum(x, axis=1)
        m = jnp.max(x, axis=1)
        s_sc[k] = jnp.sum(s, axis=-1, keepdims=True)
        m_sc[k] = jnp.max(m, axis=-1, keepdims=True)

    _big(xoa_ref, osa_sum, osa_max)
    _big(xob_ref, osb_sum, osb_max)

    @pl.when(k < ns)
    def _small():
        def _rows(ref, s_sc, m_sc):
            xs = ref[...]
            s_sc[k] = jnp.sum(xs, axis=-1, keepdims=True)
            m_sc[k] = jnp.max(xs, axis=-1, keepdims=True)

        _rows(xe_ref, es_sum, es_max)
        _rows(xx_ref, xs_sum, xs_max)

    @pl.when(k == nco - 1)
    def _finalize():
        c_x, c_e, c_o = widths
        half = bloc * c_o // 2
        o_s = jnp.concatenate(
            [osa_sum[...].reshape(1, half), osb_sum[...].reshape(1, half)],
            axis=0)
        o_m = jnp.concatenate(
            [osa_max[...].reshape(1, half), osb_max[...].reshape(1, half)],
            axis=0)
        oa = o_s.reshape(bloc, c_o) * inv_o
        om = o_m.reshape(bloc, c_o)
        ea = es_sum[...].reshape(bloc, c_e) * inv_e
        em = es_max[...].reshape(bloc, c_e)
        xa = xs_sum[...].reshape(bloc, c_x) * inv_x
        xm = xs_max[...].reshape(bloc, c_x)

        offs = [0, c_x, 2 * c_x, 2 * c_x + c_e, 2 * c_x + 2 * c_e,
                2 * c_x + 2 * c_e + c_o, 2 * c_x + 2 * c_e + 2 * c_o]
        feats = (xa, xm, ea, em, oa, om)
        h = b1_ref[...]
        for f, lo, hi in zip(feats, offs[:-1], offs[1:]):
            h = h + jnp.dot(f, w1_ref[lo:hi, :],
                            preferred_element_type=jnp.float32)
        h = jnp.dot(h, w2_ref[...], preferred_element_type=jnp.float32) + b2_ref[...]
        y = jnp.dot(h, w3_ref[...], preferred_element_type=jnp.float32) + b3_ref[...]
        out_ref[...] = y.reshape(1, bloc, y.shape[-1])


def _pick_ns(small_rows, limit=16):
    for n in range(limit, 0, -1):
        if all(r % n == 0 and (r // n) % 8 == 0 for r in small_rows):
            return n
    return 1


def kernel(x4_1, encoder_output, out_feature, w1, b1, w2, b2, w3, b3):
    B = int(x4_1.shape[0])
    cores = 2
    assert B % cores == 0
    bloc = B // cores

    def _flatten(x):
        c = int(x.shape[1])
        s = 1
        for d in x.shape[2:]:
            s *= int(d)
        return x.reshape(B * c, s), c, s

    xo2, c_o, s_o = _flatten(out_feature)
    xe, c_e, s_e = _flatten(encoder_output)
    xx, c_x, s_x = _flatten(x4_1)

    assert s_o % _LANES == 0
    xo = xo2.reshape(B * c_o, s_o // _LANES, _LANES)

    ro, re, rx = B * c_o // cores, B * c_e // cores, B * c_x // cores
    assert ro % 2 == 0
    nco = ro // 2
    ns = _pick_ns((re, rx), limit=max(1, nco - 1))
    assert ns < nco
    re_b, rx_b = re // ns, rx // ns

    n_out = int(w3.shape[1])
    weights = (w1, b1, w2, b2, w3, b3)

    def _const_spec(a):
        return pl.BlockSpec(a.shape, lambda i, k: (0,) * a.ndim)

    body = functools.partial(
        _fused_body, nco=nco, ns=ns, bloc=bloc, widths=(c_x, c_e, c_o),
        inv_o=1.0 / s_o, inv_e=1.0 / s_e, inv_x=1.0 / s_x)

    out = pl.pallas_call(
        body,
        out_shape=jax.ShapeDtypeStruct((cores, bloc, n_out), jnp.float32),
        grid=(cores, nco),
        in_specs=[
            pl.BlockSpec(
                (1, s_o // _LANES, _LANES),
                lambda i, k, _r=ro: (i * _r + k, 0, 0)),
            pl.BlockSpec(
                (1, s_o // _LANES, _LANES),
                lambda i, k, _r=ro, _n=nco: (i * _r + _n + k, 0, 0)),
            pl.BlockSpec(
                (re_b, s_e),
                lambda i, k, _s=ns: (i * _s + jnp.minimum(k, _s - 1), 0)),
            pl.BlockSpec(
                (rx_b, s_x),
                lambda i, k, _s=ns: (i * _s + jnp.minimum(k, _s - 1), 0)),
            *[_const_spec(a) for a in weights],
        ],
        out_specs=pl.BlockSpec((1, bloc, n_out), lambda i, k: (i, 0, 0)),
        scratch_shapes=[
            pltpu.VMEM((nco, 1, 1), jnp.float32),
            pltpu.VMEM((nco, 1, 1), jnp.float32),
            pltpu.VMEM((nco, 1, 1), jnp.float32),
            pltpu.VMEM((nco, 1, 1), jnp.float32),
            pltpu.VMEM((ns, re_b, 1), jnp.float32),
            pltpu.VMEM((ns, re_b, 1), jnp.float32),
            pltpu.VMEM((ns, rx_b, 1), jnp.float32),
            pltpu.VMEM((ns, rx_b, 1), jnp.float32),
        ],
        compiler_params=pltpu.CompilerParams(
            dimension_semantics=("parallel", "arbitrary"),
            vmem_limit_bytes=56 * _MIB,
        ),
    )(xo, xo, xe, xx, *weights)
    return out.reshape(B, n_out)


# speedup vs baseline: 1.0020x; 1.0020x over previous
import functools

import jax
import jax.numpy as jnp
from jax.experimental import pallas as pl
from jax.experimental.pallas import tpu as pltpu

_MIB = 1024 * 1024
_LANES = 128


def _fused_body(xo_ref, xe_ref, xx_ref,
                w1_ref, b1_ref, w2_ref, b2_ref, w3_ref, b3_ref,
                out_ref,
                os_sum, os_max, es_sum, es_max, xs_sum, xs_max,
                *, nco, ns, bloc, widths, inv_o, inv_e, inv_x):
    k = pl.program_id(1)

    x = xo_ref[...]
    s = jnp.sum(x, axis=1)
    m = jnp.max(x, axis=1)
    os_sum[k] = jnp.sum(s, axis=-1, keepdims=True)
    os_max[k] = jnp.max(m, axis=-1, keepdims=True)

    @pl.when(k < ns)
    def _small():
        def _rows(ref, s_sc, m_sc):
            xs = ref[...]
            s_sc[k] = jnp.sum(xs, axis=-1, keepdims=True)
            m_sc[k] = jnp.max(xs, axis=-1, keepdims=True)

        _rows(xe_ref, es_sum, es_max)
        _rows(xx_ref, xs_sum, xs_max)

    @pl.when(k == nco - 1)
    def _finalize():
        c_x, c_e, c_o = widths
        oa = os_sum[...].reshape(bloc, c_o) * inv_o
        om = os_max[...].reshape(bloc, c_o)
        ea = es_sum[...].reshape(bloc, c_e) * inv_e
        em = es_max[...].reshape(bloc, c_e)
        xa = xs_sum[...].reshape(bloc, c_x) * inv_x
        xm = xs_max[...].reshape(bloc, c_x)

        offs = [0, c_x, 2 * c_x, 2 * c_x + c_e, 2 * c_x + 2 * c_e,
                2 * c_x + 2 * c_e + c_o, 2 * c_x + 2 * c_e + 2 * c_o]
        feats = (xa, xm, ea, em, oa, om)
        h = b1_ref[...]
        for f, lo, hi in zip(feats, offs[:-1], offs[1:]):
            h = h + jnp.dot(f, w1_ref[lo:hi, :],
                            preferred_element_type=jnp.float32)
        h = jnp.dot(h, w2_ref[...], preferred_element_type=jnp.float32) + b2_ref[...]
        y = jnp.dot(h, w3_ref[...], preferred_element_type=jnp.float32) + b3_ref[...]
        out_ref[...] = y.reshape(1, bloc, y.shape[-1])


def _pick_ns(small_rows, limit=16):
    for n in range(limit, 0, -1):
        if all(r % n == 0 and (r // n) % 8 == 0 for r in small_rows):
            return n
    return 1


def kernel(x4_1, encoder_output, out_feature, w1, b1, w2, b2, w3, b3):
    B = int(x4_1.shape[0])
    cores = 2
    assert B % cores == 0
    bloc = B // cores

    def _flatten(x):
        c = int(x.shape[1])
        s = 1
        for d in x.shape[2:]:
            s *= int(d)
        return x.reshape(B * c, s), c, s

    xo2, c_o, s_o = _flatten(out_feature)
    xe, c_e, s_e = _flatten(encoder_output)
    xx, c_x, s_x = _flatten(x4_1)

    assert s_o % _LANES == 0
    xo = xo2.reshape(B * c_o, s_o // _LANES, _LANES)

    ro, re, rx = B * c_o // cores, B * c_e // cores, B * c_x // cores
    rpb = 2
    assert ro % rpb == 0
    nco = ro // rpb
    ns = _pick_ns((re, rx), limit=max(1, nco - 1))
    assert ns < nco
    re_b, rx_b = re // ns, rx // ns

    n_out = int(w3.shape[1])
    weights = (w1, b1, w2, b2, w3, b3)

    def _const_spec(a):
        return pl.BlockSpec(a.shape, lambda i, k: (0,) * a.ndim)

    body = functools.partial(
        _fused_body, nco=nco, ns=ns, bloc=bloc, widths=(c_x, c_e, c_o),
        inv_o=1.0 / s_o, inv_e=1.0 / s_e, inv_x=1.0 / s_x)

    out = pl.pallas_call(
        body,
        out_shape=jax.ShapeDtypeStruct((cores, bloc, n_out), jnp.float32),
        grid=(cores, nco),
        in_specs=[
            pl.BlockSpec(
                (rpb, s_o // _LANES, _LANES),
                lambda i, k, _n=nco: (i * _n + k, 0, 0)),
            pl.BlockSpec(
                (re_b, s_e),
                lambda i, k, _s=ns: (i * _s + jnp.minimum(k, _s - 1), 0)),
            pl.BlockSpec(
                (rx_b, s_x),
                lambda i, k, _s=ns: (i * _s + jnp.minimum(k, _s - 1), 0)),
            *[_const_spec(a) for a in weights],
        ],
        out_specs=pl.BlockSpec((1, bloc, n_out), lambda i, k: (i, 0, 0)),
        scratch_shapes=[
            pltpu.VMEM((nco, rpb, 1), jnp.float32),
            pltpu.VMEM((nco, rpb, 1), jnp.float32),
            pltpu.VMEM((ns, re_b, 1), jnp.float32),
            pltpu.VMEM((ns, re_b, 1), jnp.float32),
            pltpu.VMEM((ns, rx_b, 1), jnp.float32),
            pltpu.VMEM((ns, rx_b, 1), jnp.float32),
        ],
        compiler_params=pltpu.CompilerParams(
            dimension_semantics=("parallel", "arbitrary"),
            vmem_limit_bytes=56 * _MIB,
        ),
    )(xo, xe, xx, *weights)
    return out.reshape(B, n_out)
